# baseline (speedup 1.0000x reference)
import jax
import jax.numpy as jnp
from jax import lax
from jax.experimental import pallas as pl
from jax.experimental.pallas import tpu as pltpu
from jax.experimental.pallas import tpu_sc as plsc

N = 10000
E = 320000
D = 128
H = 128
HH = H // 2
C = 40
L = 4

NC = 2
NS = 16
NW = NC * NS

EW = E // NW
K = 128
CH = -(-EW // K)
EWP = CH * K
CHE = CH - (CH % 4)
DUMMY = N
N_PAD = 10240
RPT = N_PAD // NS
WK = 128
WCH = RPT // WK

_BLK = 1000

_mesh = plsc.VectorSubcoreMesh(
    core_axis_name="c", subcore_axis_name="s", num_cores=NC, num_subcores=NS)


def _sc_agg_body(h0_hbm, h1_hbm, src_hbm, dst_hbm, z64_hbm,
                 s0_hbm, s1_hbm,
                 src_v, dst_v, buf0, buf1, buf2, buf3, zbuf,
                 acc_shr, gs0, gs1, gs2, gs3, ss0, ss1, ss2, ss3):
    cid = lax.axis_index("c")
    sid = lax.axis_index("s")
    wid = cid * NS + sid
    base = sid * RPT
    pltpu.sync_copy(src_hbm.at[wid], src_v)
    pltpu.sync_copy(dst_hbm.at[wid], dst_v)
    pltpu.sync_copy(z64_hbm, zbuf)
    bufs = (buf0, buf1, buf2, buf3)
    gsems = (gs0, gs1, gs2, gs3)
    ssems = (ss0, ss1, ss2, ss3)

    for cp, (h_hbm, out_hbm) in enumerate(((h0_hbm, s0_hbm),
                                           (h1_hbm, s1_hbm))):
        for w in range(WCH):
            pltpu.sync_copy(zbuf, acc_shr.at[pl.ds(base + w * WK, WK)])
        plsc.subcore_barrier()

        pltpu.async_copy(h_hbm.at[src_v.at[0]], buf0, gs0)
        pltpu.async_copy(h_hbm.at[src_v.at[1]], buf1, gs1)

        @pl.loop(0, CHE, step=4)
        def _(j):
            for b in range(4):
                jj = j + b
                buf, gsem, ssem = bufs[b], gsems[b], ssems[b]
                pltpu.make_async_copy(h_hbm.at[src_v.at[jj]], buf,
                                      gsem).wait()
                pltpu.async_copy(buf, acc_shr.at[dst_v.at[jj]], ssem,
                                 add=True)
                nb = (b + 2) % 4

                @pl.when(jj >= 2)
                def _():
                    pltpu.make_async_copy(
                        bufs[nb], acc_shr.at[dst_v.at[jj - 2]],
                        ssems[nb]).wait()

                @pl.when(jj + 2 < CH)
                def _():
                    pltpu.async_copy(h_hbm.at[src_v.at[jj + 2]], bufs[nb],
                                     gsems[nb])

        for jj in (CHE - 2, CHE - 1):
            b = jj % 4
            pltpu.make_async_copy(bufs[b], acc_shr.at[dst_v.at[jj]],
                                  ssems[b]).wait()
        for jj in range(CHE, CH):
            b = jj % 4
            if jj >= CHE + 2:
                pltpu.async_copy(h_hbm.at[src_v.at[jj]], bufs[b],
                                 gsems[b])
            pltpu.make_async_copy(h_hbm.at[src_v.at[jj]], bufs[b],
                                  gsems[b]).wait()
            pltpu.sync_copy(bufs[b], acc_shr.at[dst_v.at[jj]], add=True)

        plsc.subcore_barrier()
        for w in range(WCH):
            r = base + w * WK
            pltpu.sync_copy(acc_shr.at[pl.ds(r, WK)], zbuf)
            pltpu.sync_copy(zbuf, out_hbm.at[cid, pl.ds(r, WK)])
        if cp == 0:
            pltpu.sync_copy(z64_hbm, zbuf)
            plsc.subcore_barrier()


def _sc_deg_body(dst_hbm, o16_hbm, z16_hbm,
                 deg_hbm,
                 dst_v, ones_v, d16_v,
                 dacc_shr, sem0):
    cid = lax.axis_index("c")
    sid = lax.axis_index("s")
    wid = cid * NS + sid
    base = sid * RPT
    pltpu.sync_copy(dst_hbm.at[wid], dst_v)
    pltpu.sync_copy(z16_hbm, d16_v)
    for w in range(WCH):
        pltpu.sync_copy(d16_v, dacc_shr.at[pl.ds(base + w * WK, WK)])
    pltpu.sync_copy(o16_hbm, ones_v)
    plsc.subcore_barrier()

    @pl.loop(0, CH)
    def _(j):
        pltpu.sync_copy(ones_v, dacc_shr.at[dst_v.at[j]], add=True)

    plsc.subcore_barrier()
    for w in range(WCH):
        r = base + w * WK
        pltpu.sync_copy(dacc_shr.at[pl.ds(r, WK)], d16_v)
        pltpu.sync_copy(d16_v, deg_hbm.at[cid, pl.ds(r, WK)])


_sc_agg = pl.kernel(
    _sc_agg_body,
    out_type=(jax.ShapeDtypeStruct((NC, N_PAD, HH), jnp.float32),
              jax.ShapeDtypeStruct((NC, N_PAD, HH), jnp.float32)),
    mesh=_mesh,
    scratch_types=[
        pltpu.VMEM((CH, K), jnp.int32),
        pltpu.VMEM((CH, K), jnp.int32),
        pltpu.VMEM((K, HH), jnp.float32),
        pltpu.VMEM((K, HH), jnp.float32),
        pltpu.VMEM((K, HH), jnp.float32),
        pltpu.VMEM((K, HH), jnp.float32),
        pltpu.VMEM((WK, HH), jnp.float32),
        pltpu.VMEM_SHARED((N_PAD, HH), jnp.float32),
        pltpu.SemaphoreType.DMA,
        pltpu.SemaphoreType.DMA,
        pltpu.SemaphoreType.DMA,
        pltpu.SemaphoreType.DMA,
        pltpu.SemaphoreType.DMA,
        pltpu.SemaphoreType.DMA,
        pltpu.SemaphoreType.DMA,
        pltpu.SemaphoreType.DMA,
    ],
    compiler_params=pltpu.CompilerParams(use_tc_tiling_on_sc=False),
)

_sc_deg = pl.kernel(
    _sc_deg_body,
    out_type=jax.ShapeDtypeStruct((NC, N_PAD, 16), jnp.float32),
    mesh=_mesh,
    scratch_types=[
        pltpu.VMEM((CH, K), jnp.int32),
        pltpu.VMEM((K, 16), jnp.float32),
        pltpu.VMEM((WK, 16), jnp.float32),
        pltpu.VMEM_SHARED((N_PAD, 16), jnp.float32),
        pltpu.SemaphoreType.DMA,
    ],
    compiler_params=pltpu.CompilerParams(use_tc_tiling_on_sc=False),
)


def _dot_t(a, w):
    return lax.dot_general(a, w, (((1,), (1,)), ((), ())),
                           preferred_element_type=jnp.float32)


def _split_out(o0_ref, o1_ref, val):
    o0_ref[...] = val[:, :HH]
    o1_ref[...] = val[:, HH:]


def _proj_body(x_ref, w_ref, b_ref, o0_ref, o1_ref):
    t = jnp.maximum(_dot_t(x_ref[...], w_ref[...]) + b_ref[...], 0.0)
    _split_out(o0_ref, o1_ref, t)


def _proj(x, Win, b_in):
    full = lambda s: pl.BlockSpec(s, lambda i: (0, 0))
    half = pl.BlockSpec((_BLK, HH), lambda i: (i, 0))
    return pl.pallas_call(
        _proj_body,
        grid=(N // _BLK,),
        in_specs=[pl.BlockSpec((_BLK, D), lambda i: (i, 0)),
                  full((H, D)), full((1, H))],
        out_specs=(half, half),
        out_shape=(jax.ShapeDtypeStruct((N, HH), jnp.float32),
                   jax.ShapeDtypeStruct((N, HH), jnp.float32)),
    )(x, Win, b_in.reshape(1, -1))


def _layer_body(s0_ref, s1_ref, d_ref, h0_ref, h1_ref,
                wl_ref, bl_ref, wr_ref, ws_ref, bs_ref, g_ref, bt_ref,
                o0_ref, o1_ref):
    s = jnp.concatenate([s0_ref[0] + s0_ref[1], s1_ref[0] + s1_ref[1]],
                        axis=1)
    degc = d_ref[0] + d_ref[1]
    deg = degc[:, 0:1]
    mean = s * (1.0 / jnp.maximum(deg, 1.0))
    h = jnp.concatenate([h0_ref[...], h1_ref[...]], axis=1)
    t = _dot_t(mean, wl_ref[...]) + bl_ref[...] + _dot_t(h, wr_ref[...])
    t = t * g_ref[...] + bt_ref[...]
    res = jnp.maximum(t, 0.0) + _dot_t(h, ws_ref[...]) + bs_ref[...]
    _split_out(o0_ref, o1_ref, res)


def _layer(s0, s1, degs, h0, h1, Wl, bl, Wr, Ws, bs, gscale, beta):
    full = lambda s: pl.BlockSpec(s, lambda i: tuple(0 for _ in s))
    half = pl.BlockSpec((_BLK, HH), lambda i: (i, 0))
    shalf = pl.BlockSpec((NC, _BLK, HH), lambda i: (0, i, 0))
    return pl.pallas_call(
        _layer_body,
        grid=(N // _BLK,),
        in_specs=[
            shalf, shalf,
            pl.BlockSpec((NC, _BLK, 16), lambda i: (0, i, 0)),
            half, half,
            full((H, H)), full((1, H)), full((H, H)),
            full((H, H)), full((1, H)),
            full((1, H)), full((1, H)),
        ],
        out_specs=(half, half),
        out_shape=(jax.ShapeDtypeStruct((N, HH), jnp.float32),
                   jax.ShapeDtypeStruct((N, HH), jnp.float32)),
    )(s0, s1, degs, h0, h1, Wl, bl.reshape(1, -1), Wr, Ws,
      bs.reshape(1, -1), gscale.reshape(1, -1), beta.reshape(1, -1))


def _final_body(h0_ref, h1_ref, w1_ref, b1_ref, w2_ref, b2_ref, wo_ref,
                bo_ref, out_ref):
    h = jnp.concatenate([h0_ref[...], h1_ref[...]], axis=1)
    t = jnp.maximum(_dot_t(h, w1_ref[...]) + b1_ref[...], 0.0)
    t = _dot_t(t, w2_ref[...]) + b2_ref[...]
    o = _dot_t(t, wo_ref[...]) + bo_ref[...]
    m = jnp.max(o, axis=1, keepdims=True)
    lse = m + jnp.log(jnp.sum(jnp.exp(o - m), axis=1, keepdims=True))
    out_ref[...] = o - lse


def _final_stage(h0, h1, W1, b1, W2, b2, Wout, bout):
    full = lambda s: pl.BlockSpec(s, lambda i: (0, 0))
    half = pl.BlockSpec((_BLK, HH), lambda i: (i, 0))
    return pl.pallas_call(
        _final_body,
        grid=(N // _BLK,),
        in_specs=[
            half, half,
            full((2 * H, H)), full((1, 2 * H)),
            full((H, 2 * H)), full((1, H)),
            full((C, H)), full((1, C)),
        ],
        out_specs=pl.BlockSpec((_BLK, C), lambda i: (i, 0)),
        out_shape=jax.ShapeDtypeStruct((N, C), jnp.float32),
    )(h0, h1, W1, b1.reshape(1, -1), W2, b2.reshape(1, -1),
      Wout, bout.reshape(1, -1))


def kernel(x, edge_index, Win, b_in, convWl, convbl, convWr, skipW, skipb,
           gamma, beta, W1, b1, W2, b2, Wout, bout):
    pad = EWP - EW
    srcp = jnp.pad(edge_index[0].reshape(NW, EW),
                   ((0, 0), (0, pad))).reshape(NW, CH, K)
    dstp = jnp.pad(edge_index[1].reshape(NW, EW), ((0, 0), (0, pad)),
                   constant_values=DUMMY).reshape(NW, CH, K)
    z64 = jnp.zeros((WK, HH), jnp.float32)
    o16 = jnp.ones((K, 16), jnp.float32)
    z16 = jnp.zeros((WK, 16), jnp.float32)
    bn = 1.0 / jnp.sqrt(1.0 + 1e-5)

    h0, h1 = _proj(x, Win, b_in)
    degs = _sc_deg(dstp, o16, z16)
    for i in range(L):
        s0, s1 = _sc_agg(h0, h1, srcp, dstp, z64)
        h0, h1 = _layer(s0, s1, degs, h0, h1, convWl[i], convbl[i],
                        convWr[i], skipW[i], skipb[i], gamma[i] * bn, beta[i])
    return _final_stage(h0, h1, W1, b1, W2, b2, Wout, bout)


# speedup vs baseline: 1.7235x; 1.7235x over previous
import jax
import jax.numpy as jnp
from jax import lax
from jax.experimental import pallas as pl
from jax.experimental.pallas import tpu as pltpu
from jax.experimental.pallas import tpu_sc as plsc

N = 10000
E = 320000
D = 128
H = 128
HH = H // 2
C = 40
L = 4

NC = 2
NS = 16
NW = NC * NS

K = 80
CH = E // (K * NW)
NBUF = 6
LEAD = 3
CHE = CH - (CH % NBUF)
N_PAD = 10240
RPT = N_PAD // NS
WK = 128
WCH = RPT // WK

_BLK = 1000

_mesh = plsc.VectorSubcoreMesh(
    core_axis_name="c", subcore_axis_name="s", num_cores=NC, num_subcores=NS)


def _sc_agg_body(h0_hbm, h1_hbm, src_hbm, dst_hbm, z64_hbm,
                 s0_hbm, s1_hbm,
                 src_v, dst_v, zbuf, *bufs_and_sems):
    bufs = bufs_and_sems[:NBUF]
    acc_shr = bufs_and_sems[NBUF]
    gsems = bufs_and_sems[NBUF + 1:2 * NBUF + 1]
    ssems = bufs_and_sems[2 * NBUF + 1:]
    cid = lax.axis_index("c")
    sid = lax.axis_index("s")
    wid = cid * NS + sid
    base = sid * RPT
    pltpu.sync_copy(src_hbm.at[wid], src_v)
    pltpu.sync_copy(dst_hbm.at[wid], dst_v)
    pltpu.sync_copy(z64_hbm, zbuf)

    for cp, (h_hbm, out_hbm) in enumerate(((h0_hbm, s0_hbm),
                                           (h1_hbm, s1_hbm))):
        for w in range(WCH):
            pltpu.sync_copy(zbuf, acc_shr.at[pl.ds(base + w * WK, WK)])
        plsc.subcore_barrier()

        for m in range(LEAD):
            pltpu.async_copy(h_hbm.at[src_v.at[m]], bufs[m], gsems[m])

        @pl.loop(0, CHE, step=NBUF)
        def _(j):
            for b in range(NBUF):
                jj = j + b
                pltpu.make_async_copy(h_hbm.at[src_v.at[jj]], bufs[b],
                                      gsems[b]).wait()
                pltpu.async_copy(bufs[b], acc_shr.at[dst_v.at[jj]],
                                 ssems[b], add=True)
                nb = (b + LEAD) % NBUF

                @pl.when(jj >= LEAD)
                def _():
                    pltpu.make_async_copy(
                        bufs[nb], acc_shr.at[dst_v.at[jj - LEAD]],
                        ssems[nb]).wait()

                @pl.when(jj + LEAD < CH)
                def _():
                    pltpu.async_copy(h_hbm.at[src_v.at[jj + LEAD]],
                                     bufs[nb], gsems[nb])

        for jj in range(CHE - LEAD, CHE):
            b = jj % NBUF
            pltpu.make_async_copy(bufs[b], acc_shr.at[dst_v.at[jj]],
                                  ssems[b]).wait()
        for jj in range(CHE, CH):
            b = jj % NBUF
            if jj >= CHE + LEAD:
                pltpu.async_copy(h_hbm.at[src_v.at[jj]], bufs[b],
                                 gsems[b])
            pltpu.make_async_copy(h_hbm.at[src_v.at[jj]], bufs[b],
                                  gsems[b]).wait()
            pltpu.sync_copy(bufs[b], acc_shr.at[dst_v.at[jj]], add=True)

        plsc.subcore_barrier()
        for w in range(WCH):
            r = base + w * WK
            pltpu.sync_copy(acc_shr.at[pl.ds(r, WK)], zbuf)
            pltpu.sync_copy(zbuf, out_hbm.at[cid, pl.ds(r, WK)])
        if cp == 0:
            pltpu.sync_copy(z64_hbm, zbuf)
            plsc.subcore_barrier()


def _sc_deg_body(dst_hbm, o16_hbm, z16_hbm,
                 deg_hbm,
                 dst_v, ones_v, d16_v,
                 dacc_shr, sem0):
    cid = lax.axis_index("c")
    sid = lax.axis_index("s")
    wid = cid * NS + sid
    base = sid * RPT
    pltpu.sync_copy(dst_hbm.at[wid], dst_v)
    pltpu.sync_copy(z16_hbm, d16_v)
    for w in range(WCH):
        pltpu.sync_copy(d16_v, dacc_shr.at[pl.ds(base + w * WK, WK)])
    pltpu.sync_copy(o16_hbm, ones_v)
    plsc.subcore_barrier()

    @pl.loop(0, CH)
    def _(j):
        pltpu.sync_copy(ones_v, dacc_shr.at[dst_v.at[j]], add=True)

    plsc.subcore_barrier()
    for w in range(WCH):
        r = base + w * WK
        pltpu.sync_copy(dacc_shr.at[pl.ds(r, WK)], d16_v)
        pltpu.sync_copy(d16_v, deg_hbm.at[cid, pl.ds(r, WK)])


_sc_agg = pl.kernel(
    _sc_agg_body,
    out_type=(jax.ShapeDtypeStruct((NC, N_PAD, HH), jnp.float32),
              jax.ShapeDtypeStruct((NC, N_PAD, HH), jnp.float32)),
    mesh=_mesh,
    scratch_types=(
        [pltpu.VMEM((CH, K), jnp.int32),
         pltpu.VMEM((CH, K), jnp.int32),
         pltpu.VMEM((WK, HH), jnp.float32)]
        + [pltpu.VMEM((K, HH), jnp.float32)] * NBUF
        + [pltpu.VMEM_SHARED((N_PAD, HH), jnp.float32)]
        + [pltpu.SemaphoreType.DMA] * (2 * NBUF)
    ),
    compiler_params=pltpu.CompilerParams(use_tc_tiling_on_sc=False),
)

_sc_deg = pl.kernel(
    _sc_deg_body,
    out_type=jax.ShapeDtypeStruct((NC, N_PAD, 16), jnp.float32),
    mesh=_mesh,
    scratch_types=[
        pltpu.VMEM((CH, K), jnp.int32),
        pltpu.VMEM((K, 16), jnp.float32),
        pltpu.VMEM((WK, 16), jnp.float32),
        pltpu.VMEM_SHARED((N_PAD, 16), jnp.float32),
        pltpu.SemaphoreType.DMA,
    ],
    compiler_params=pltpu.CompilerParams(use_tc_tiling_on_sc=False),
)


def _dot_t(a, w):
    return lax.dot_general(a, w, (((1,), (1,)), ((), ())),
                           preferred_element_type=jnp.float32)


def _split_out(o0_ref, o1_ref, val):
    o0_ref[...] = val[:, :HH]
    o1_ref[...] = val[:, HH:]


def _proj_body(x_ref, w_ref, b_ref, o0_ref, o1_ref):
    t = jnp.maximum(_dot_t(x_ref[...], w_ref[...]) + b_ref[...], 0.0)
    _split_out(o0_ref, o1_ref, t)


def _proj(x, Win, b_in):
    full = lambda s: pl.BlockSpec(s, lambda i: (0, 0))
    half = pl.BlockSpec((_BLK, HH), lambda i: (i, 0))
    return pl.pallas_call(
        _proj_body,
        grid=(N // _BLK,),
        in_specs=[pl.BlockSpec((_BLK, D), lambda i: (i, 0)),
                  full((H, D)), full((1, H))],
        out_specs=(half, half),
        out_shape=(jax.ShapeDtypeStruct((N, HH), jnp.float32),
                   jax.ShapeDtypeStruct((N, HH), jnp.float32)),
    )(x, Win, b_in.reshape(1, -1))


def _layer_body(s0_ref, s1_ref, d_ref, h0_ref, h1_ref,
                wl_ref, bl_ref, wr_ref, ws_ref, bs_ref, g_ref, bt_ref,
                o0_ref, o1_ref):
    s = jnp.concatenate([s0_ref[0] + s0_ref[1], s1_ref[0] + s1_ref[1]],
                        axis=1)
    degc = d_ref[0] + d_ref[1]
    deg = degc[:, 0:1]
    mean = s * (1.0 / jnp.maximum(deg, 1.0))
    h = jnp.concatenate([h0_ref[...], h1_ref[...]], axis=1)
    t = _dot_t(mean, wl_ref[...]) + bl_ref[...] + _dot_t(h, wr_ref[...])
    t = t * g_ref[...] + bt_ref[...]
    res = jnp.maximum(t, 0.0) + _dot_t(h, ws_ref[...]) + bs_ref[...]
    _split_out(o0_ref, o1_ref, res)


def _layer(s0, s1, degs, h0, h1, Wl, bl, Wr, Ws, bs, gscale, beta):
    full = lambda s: pl.BlockSpec(s, lambda i: tuple(0 for _ in s))
    half = pl.BlockSpec((_BLK, HH), lambda i: (i, 0))
    shalf = pl.BlockSpec((NC, _BLK, HH), lambda i: (0, i, 0))
    return pl.pallas_call(
        _layer_body,
        grid=(N // _BLK,),
        in_specs=[
            shalf, shalf,
            pl.BlockSpec((NC, _BLK, 16), lambda i: (0, i, 0)),
            half, half,
            full((H, H)), full((1, H)), full((H, H)),
            full((H, H)), full((1, H)),
            full((1, H)), full((1, H)),
        ],
        out_specs=(half, half),
        out_shape=(jax.ShapeDtypeStruct((N, HH), jnp.float32),
                   jax.ShapeDtypeStruct((N, HH), jnp.float32)),
    )(s0, s1, degs, h0, h1, Wl, bl.reshape(1, -1), Wr, Ws,
      bs.reshape(1, -1), gscale.reshape(1, -1), beta.reshape(1, -1))


def _final_body(h0_ref, h1_ref, w1_ref, b1_ref, w2_ref, b2_ref, wo_ref,
                bo_ref, out_ref):
    h = jnp.concatenate([h0_ref[...], h1_ref[...]], axis=1)
    t = jnp.maximum(_dot_t(h, w1_ref[...]) + b1_ref[...], 0.0)
    t = _dot_t(t, w2_ref[...]) + b2_ref[...]
    o = _dot_t(t, wo_ref[...]) + bo_ref[...]
    m = jnp.max(o, axis=1, keepdims=True)
    lse = m + jnp.log(jnp.sum(jnp.exp(o - m), axis=1, keepdims=True))
    out_ref[...] = o - lse


def _final_stage(h0, h1, W1, b1, W2, b2, Wout, bout):
    full = lambda s: pl.BlockSpec(s, lambda i: (0, 0))
    half = pl.BlockSpec((_BLK, HH), lambda i: (i, 0))
    return pl.pallas_call(
        _final_body,
        grid=(N // _BLK,),
        in_specs=[
            half, half,
            full((2 * H, H)), full((1, 2 * H)),
            full((H, 2 * H)), full((1, H)),
            full((C, H)), full((1, C)),
        ],
        out_specs=pl.BlockSpec((_BLK, C), lambda i: (i, 0)),
        out_shape=jax.ShapeDtypeStruct((N, C), jnp.float32),
    )(h0, h1, W1, b1.reshape(1, -1), W2, b2.reshape(1, -1),
      Wout, bout.reshape(1, -1))


def kernel(x, edge_index, Win, b_in, convWl, convbl, convWr, skipW, skipb,
           gamma, beta, W1, b1, W2, b2, Wout, bout):
    srcp = edge_index[0].reshape(NW, CH, K)
    dstp = edge_index[1].reshape(NW, CH, K)
    z64 = jnp.zeros((WK, HH), jnp.float32)
    o16 = jnp.ones((K, 16), jnp.float32)
    z16 = jnp.zeros((WK, 16), jnp.float32)
    bn = 1.0 / jnp.sqrt(1.0 + 1e-5)

    h0, h1 = _proj(x, Win, b_in)
    degs = _sc_deg(dstp, o16, z16)
    for i in range(L):
        s0, s1 = _sc_agg(h0, h1, srcp, dstp, z64)
        h0, h1 = _layer(s0, s1, degs, h0, h1, convWl[i], convbl[i],
                        convWr[i], skipW[i], skipb[i], gamma[i] * bn, beta[i])
    return _final_stage(h0, h1, W1, b1, W2, b2, Wout, bout)


# speedup vs baseline: 1.8144x; 1.0528x over previous
import jax
import jax.numpy as jnp
from jax import lax
from jax.experimental import pallas as pl
from jax.experimental.pallas import tpu as pltpu
from jax.experimental.pallas import tpu_sc as plsc

N = 10000
E = 320000
D = 128
H = 128
HH = H // 2
C = 40
L = 4

NC = 2
NS = 16
NW = NC * NS

K = 80
CH = E // (K * NW)
NBUF = 8
LEAD = 4
CHE = CH - (CH % NBUF)
N_PAD = 10240
RPT = N_PAD // NS
WK = 128
WCH = RPT // WK

_BLK = 1000

_mesh = plsc.VectorSubcoreMesh(
    core_axis_name="c", subcore_axis_name="s", num_cores=NC, num_subcores=NS)


def _sc_agg_body(h0_hbm, h1_hbm, src_hbm, dst_hbm, z64_hbm,
                 s0_hbm, s1_hbm,
                 src_v, dst_v, zbuf, *bufs_and_sems):
    bufs = bufs_and_sems[:NBUF]
    acc_shr = bufs_and_sems[NBUF]
    gsems = bufs_and_sems[NBUF + 1:2 * NBUF + 1]
    ssems = bufs_and_sems[2 * NBUF + 1:]
    cid = lax.axis_index("c")
    sid = lax.axis_index("s")
    wid = cid * NS + sid
    base = sid * RPT
    pltpu.sync_copy(src_hbm.at[wid], src_v)
    pltpu.sync_copy(dst_hbm.at[wid], dst_v)
    pltpu.sync_copy(z64_hbm, zbuf)

    for cp, (h_hbm, out_hbm) in enumerate(((h0_hbm, s0_hbm),
                                           (h1_hbm, s1_hbm))):
        for w in range(WCH):
            pltpu.sync_copy(zbuf, acc_shr.at[pl.ds(base + w * WK, WK)])
        plsc.subcore_barrier()

        for m in range(LEAD):
            pltpu.async_copy(h_hbm.at[src_v.at[m]], bufs[m], gsems[m])

        @pl.loop(0, CHE, step=NBUF)
        def _(j):
            for b in range(NBUF):
                jj = j + b
                pltpu.make_async_copy(h_hbm.at[src_v.at[jj]], bufs[b],
                                      gsems[b]).wait()
                pltpu.async_copy(bufs[b], acc_shr.at[dst_v.at[jj]],
                                 ssems[b], add=True)
                nb = (b + LEAD) % NBUF

                @pl.when(jj >= LEAD)
                def _():
                    pltpu.make_async_copy(
                        bufs[nb], acc_shr.at[dst_v.at[jj - LEAD]],
                        ssems[nb]).wait()

                @pl.when(jj + LEAD < CH)
                def _():
                    pltpu.async_copy(h_hbm.at[src_v.at[jj + LEAD]],
                                     bufs[nb], gsems[nb])

        for jj in range(CHE - LEAD, CHE):
            b = jj % NBUF
            pltpu.make_async_copy(bufs[b], acc_shr.at[dst_v.at[jj]],
                                  ssems[b]).wait()
        for jj in range(CHE, CH):
            b = jj % NBUF
            if jj >= CHE + LEAD:
                pltpu.async_copy(h_hbm.at[src_v.at[jj]], bufs[b],
                                 gsems[b])
            pltpu.make_async_copy(h_hbm.at[src_v.at[jj]], bufs[b],
                                  gsems[b]).wait()
            pltpu.sync_copy(bufs[b], acc_shr.at[dst_v.at[jj]], add=True)

        plsc.subcore_barrier()
        for w in range(WCH):
            r = base + w * WK
            pltpu.sync_copy(acc_shr.at[pl.ds(r, WK)], zbuf)
            pltpu.sync_copy(zbuf, out_hbm.at[cid, pl.ds(r, WK)])
        if cp == 0:
            pltpu.sync_copy(z64_hbm, zbuf)
            plsc.subcore_barrier()


def _sc_deg_body(dst_hbm, o16_hbm, z16_hbm,
                 deg_hbm,
                 dst_v, ones_v, d16_v,
                 dacc_shr, sem0):
    cid = lax.axis_index("c")
    sid = lax.axis_index("s")
    wid = cid * NS + sid
    base = sid * RPT
    pltpu.sync_copy(dst_hbm.at[wid], dst_v)
    pltpu.sync_copy(z16_hbm, d16_v)
    for w in range(WCH):
        pltpu.sync_copy(d16_v, dacc_shr.at[pl.ds(base + w * WK, WK)])
    pltpu.sync_copy(o16_hbm, ones_v)
    plsc.subcore_barrier()

    @pl.loop(0, CH)
    def _(j):
        pltpu.sync_copy(ones_v, dacc_shr.at[dst_v.at[j]], add=True)

    plsc.subcore_barrier()
    for w in range(WCH):
        r = base + w * WK
        pltpu.sync_copy(dacc_shr.at[pl.ds(r, WK)], d16_v)
        pltpu.sync_copy(d16_v, deg_hbm.at[cid, pl.ds(r, WK)])


_sc_agg = pl.kernel(
    _sc_agg_body,
    out_type=(jax.ShapeDtypeStruct((NC, N_PAD, HH), jnp.float32),
              jax.ShapeDtypeStruct((NC, N_PAD, HH), jnp.float32)),
    mesh=_mesh,
    scratch_types=(
        [pltpu.VMEM((CH, K), jnp.int32),
         pltpu.VMEM((CH, K), jnp.int32),
         pltpu.VMEM((WK, HH), jnp.float32)]
        + [pltpu.VMEM((K, HH), jnp.float32)] * NBUF
        + [pltpu.VMEM_SHARED((N_PAD, HH), jnp.float32)]
        + [pltpu.SemaphoreType.DMA] * (2 * NBUF)
    ),
    compiler_params=pltpu.CompilerParams(use_tc_tiling_on_sc=False),
)

_sc_deg = pl.kernel(
    _sc_deg_body,
    out_type=jax.ShapeDtypeStruct((NC, N_PAD, 16), jnp.float32),
    mesh=_mesh,
    scratch_types=[
        pltpu.VMEM((CH, K), jnp.int32),
        pltpu.VMEM((K, 16), jnp.float32),
        pltpu.VMEM((WK, 16), jnp.float32),
        pltpu.VMEM_SHARED((N_PAD, 16), jnp.float32),
        pltpu.SemaphoreType.DMA,
    ],
    compiler_params=pltpu.CompilerParams(use_tc_tiling_on_sc=False),
)


def _dot_t(a, w):
    return lax.dot_general(a, w, (((1,), (1,)), ((), ())),
                           preferred_element_type=jnp.float32)


def _split_out(o0_ref, o1_ref, val):
    o0_ref[...] = val[:, :HH]
    o1_ref[...] = val[:, HH:]


def _proj_body(x_ref, w_ref, b_ref, o0_ref, o1_ref):
    t = jnp.maximum(_dot_t(x_ref[...], w_ref[...]) + b_ref[...], 0.0)
    _split_out(o0_ref, o1_ref, t)


def _proj(x, Win, b_in):
    full = lambda s: pl.BlockSpec(s, lambda i: (0, 0))
    half = pl.BlockSpec((_BLK, HH), lambda i: (i, 0))
    return pl.pallas_call(
        _proj_body,
        grid=(N // _BLK,),
        in_specs=[pl.BlockSpec((_BLK, D), lambda i: (i, 0)),
                  full((H, D)), full((1, H))],
        out_specs=(half, half),
        out_shape=(jax.ShapeDtypeStruct((N, HH), jnp.float32),
                   jax.ShapeDtypeStruct((N, HH), jnp.float32)),
    )(x, Win, b_in.reshape(1, -1))


def _layer_body(s0_ref, s1_ref, d_ref, h0_ref, h1_ref,
                wl_ref, bl_ref, wr_ref, ws_ref, bs_ref, g_ref, bt_ref,
                o0_ref, o1_ref):
    s = jnp.concatenate([s0_ref[0] + s0_ref[1], s1_ref[0] + s1_ref[1]],
                        axis=1)
    degc = d_ref[0] + d_ref[1]
    deg = degc[:, 0:1]
    mean = s * (1.0 / jnp.maximum(deg, 1.0))
    h = jnp.concatenate([h0_ref[...], h1_ref[...]], axis=1)
    t = _dot_t(mean, wl_ref[...]) + bl_ref[...] + _dot_t(h, wr_ref[...])
    t = t * g_ref[...] + bt_ref[...]
    res = jnp.maximum(t, 0.0) + _dot_t(h, ws_ref[...]) + bs_ref[...]
    _split_out(o0_ref, o1_ref, res)


def _layer(s0, s1, degs, h0, h1, Wl, bl, Wr, Ws, bs, gscale, beta):
    full = lambda s: pl.BlockSpec(s, lambda i: tuple(0 for _ in s))
    half = pl.BlockSpec((_BLK, HH), lambda i: (i, 0))
    shalf = pl.BlockSpec((NC, _BLK, HH), lambda i: (0, i, 0))
    return pl.pallas_call(
        _layer_body,
        grid=(N // _BLK,),
        in_specs=[
            shalf, shalf,
            pl.BlockSpec((NC, _BLK, 16), lambda i: (0, i, 0)),
            half, half,
            full((H, H)), full((1, H)), full((H, H)),
            full((H, H)), full((1, H)),
            full((1, H)), full((1, H)),
        ],
        out_specs=(half, half),
        out_shape=(jax.ShapeDtypeStruct((N, HH), jnp.float32),
                   jax.ShapeDtypeStruct((N, HH), jnp.float32)),
    )(s0, s1, degs, h0, h1, Wl, bl.reshape(1, -1), Wr, Ws,
      bs.reshape(1, -1), gscale.reshape(1, -1), beta.reshape(1, -1))


def _final_body(h0_ref, h1_ref, w1_ref, b1_ref, w2_ref, b2_ref, wo_ref,
                bo_ref, out_ref):
    h = jnp.concatenate([h0_ref[...], h1_ref[...]], axis=1)
    t = jnp.maximum(_dot_t(h, w1_ref[...]) + b1_ref[...], 0.0)
    t = _dot_t(t, w2_ref[...]) + b2_ref[...]
    o = _dot_t(t, wo_ref[...]) + bo_ref[...]
    m = jnp.max(o, axis=1, keepdims=True)
    lse = m + jnp.log(jnp.sum(jnp.exp(o - m), axis=1, keepdims=True))
    out_ref[...] = o - lse


def _final_stage(h0, h1, W1, b1, W2, b2, Wout, bout):
    full = lambda s: pl.BlockSpec(s, lambda i: (0, 0))
    half = pl.BlockSpec((_BLK, HH), lambda i: (i, 0))
    return pl.pallas_call(
        _final_body,
        grid=(N // _BLK,),
        in_specs=[
            half, half,
            full((2 * H, H)), full((1, 2 * H)),
            full((H, 2 * H)), full((1, H)),
            full((C, H)), full((1, C)),
        ],
        out_specs=pl.BlockSpec((_BLK, C), lambda i: (i, 0)),
        out_shape=jax.ShapeDtypeStruct((N, C), jnp.float32),
    )(h0, h1, W1, b1.reshape(1, -1), W2, b2.reshape(1, -1),
      Wout, bout.reshape(1, -1))


def kernel(x, edge_index, Win, b_in, convWl, convbl, convWr, skipW, skipb,
           gamma, beta, W1, b1, W2, b2, Wout, bout):
    srcp = edge_index[0].reshape(NW, CH, K)
    dstp = edge_index[1].reshape(NW, CH, K)
    z64 = jnp.zeros((WK, HH), jnp.float32)
    o16 = jnp.ones((K, 16), jnp.float32)
    z16 = jnp.zeros((WK, 16), jnp.float32)
    bn = 1.0 / jnp.sqrt(1.0 + 1e-5)

    h0, h1 = _proj(x, Win, b_in)
    degs = _sc_deg(dstp, o16, z16)
    for i in range(L):
        s0, s1 = _sc_agg(h0, h1, srcp, dstp, z64)
        h0, h1 = _layer(s0, s1, degs, h0, h1, convWl[i], convbl[i],
                        convWr[i], skipW[i], skipb[i], gamma[i] * bn, beta[i])
    return _final_stage(h0, h1, W1, b1, W2, b2, Wout, bout)


# speedup vs baseline: 1.8734x; 1.0325x over previous
import jax
import jax.numpy as jnp
from jax import lax
from jax.experimental import pallas as pl
from jax.experimental.pallas import tpu as pltpu
from jax.experimental.pallas import tpu_sc as plsc

N = 10000
E = 320000
D = 128
H = 128
HH = H // 2
C = 40
L = 4

NC = 2
NS = 16
NW = NC * NS

K = 80
CH = E // (K * NW)
NBUF = 10
LEAD = 5
CHE = CH - (CH % NBUF)
N_PAD = 10240
RPT = N_PAD // NS
WK = 128
WCH = RPT // WK

_BLK = 1000

_mesh = plsc.VectorSubcoreMesh(
    core_axis_name="c", subcore_axis_name="s", num_cores=NC, num_subcores=NS)


def _sc_agg_body(h0_hbm, h1_hbm, src_hbm, dst_hbm, z64_hbm,
                 s0_hbm, s1_hbm,
                 src_v, dst_v, zbuf, *bufs_and_sems):
    bufs = bufs_and_sems[:NBUF]
    acc_shr = bufs_and_sems[NBUF]
    gsems = bufs_and_sems[NBUF + 1:2 * NBUF + 1]
    ssems = bufs_and_sems[2 * NBUF + 1:]
    cid = lax.axis_index("c")
    sid = lax.axis_index("s")
    wid = cid * NS + sid
    base = sid * RPT
    pltpu.sync_copy(src_hbm.at[wid], src_v)
    pltpu.sync_copy(dst_hbm.at[wid], dst_v)
    pltpu.sync_copy(z64_hbm, zbuf)

    for cp, (h_hbm, out_hbm) in enumerate(((h0_hbm, s0_hbm),
                                           (h1_hbm, s1_hbm))):
        for w in range(WCH):
            pltpu.sync_copy(zbuf, acc_shr.at[pl.ds(base + w * WK, WK)])
        plsc.subcore_barrier()

        for m in range(LEAD):
            pltpu.async_copy(h_hbm.at[src_v.at[m]], bufs[m], gsems[m])

        @pl.loop(0, CHE, step=NBUF)
        def _(j):
            for b in range(NBUF):
                jj = j + b
                pltpu.make_async_copy(h_hbm.at[src_v.at[jj]], bufs[b],
                                      gsems[b]).wait()
                pltpu.async_copy(bufs[b], acc_shr.at[dst_v.at[jj]],
                                 ssems[b], add=True)
                nb = (b + LEAD) % NBUF

                @pl.when(jj >= LEAD)
                def _():
                    pltpu.make_async_copy(
                        bufs[nb], acc_shr.at[dst_v.at[jj - LEAD]],
                        ssems[nb]).wait()

                @pl.when(jj + LEAD < CH)
                def _():
                    pltpu.async_copy(h_hbm.at[src_v.at[jj + LEAD]],
                                     bufs[nb], gsems[nb])

        for jj in range(CHE - LEAD, CHE):
            b = jj % NBUF
            pltpu.make_async_copy(bufs[b], acc_shr.at[dst_v.at[jj]],
                                  ssems[b]).wait()
        for jj in range(CHE, CH):
            b = jj % NBUF
            if jj >= CHE + LEAD:
                pltpu.async_copy(h_hbm.at[src_v.at[jj]], bufs[b],
                                 gsems[b])
            pltpu.make_async_copy(h_hbm.at[src_v.at[jj]], bufs[b],
                                  gsems[b]).wait()
            pltpu.sync_copy(bufs[b], acc_shr.at[dst_v.at[jj]], add=True)

        plsc.subcore_barrier()
        for w in range(WCH):
            r = base + w * WK
            pltpu.sync_copy(acc_shr.at[pl.ds(r, WK)], zbuf)
            pltpu.sync_copy(zbuf, out_hbm.at[cid, pl.ds(r, WK)])
        if cp == 0:
            pltpu.sync_copy(z64_hbm, zbuf)
            plsc.subcore_barrier()


def _sc_deg_body(dst_hbm, o16_hbm, z16_hbm,
                 deg_hbm,
                 dst_v, ones_v, d16_v,
                 dacc_shr, sem0):
    cid = lax.axis_index("c")
    sid = lax.axis_index("s")
    wid = cid * NS + sid
    base = sid * RPT
    pltpu.sync_copy(dst_hbm.at[wid], dst_v)
    pltpu.sync_copy(z16_hbm, d16_v)
    for w in range(WCH):
        pltpu.sync_copy(d16_v, dacc_shr.at[pl.ds(base + w * WK, WK)])
    pltpu.sync_copy(o16_hbm, ones_v)
    plsc.subcore_barrier()

    @pl.loop(0, CH)
    def _(j):
        pltpu.sync_copy(ones_v, dacc_shr.at[dst_v.at[j]], add=True)

    plsc.subcore_barrier()
    for w in range(WCH):
        r = base + w * WK
        pltpu.sync_copy(dacc_shr.at[pl.ds(r, WK)], d16_v)
        pltpu.sync_copy(d16_v, deg_hbm.at[cid, pl.ds(r, WK)])


_sc_agg = pl.kernel(
    _sc_agg_body,
    out_type=(jax.ShapeDtypeStruct((NC, N_PAD, HH), jnp.float32),
              jax.ShapeDtypeStruct((NC, N_PAD, HH), jnp.float32)),
    mesh=_mesh,
    scratch_types=(
        [pltpu.VMEM((CH, K), jnp.int32),
         pltpu.VMEM((CH, K), jnp.int32),
         pltpu.VMEM((WK, HH), jnp.float32)]
        + [pltpu.VMEM((K, HH), jnp.float32)] * NBUF
        + [pltpu.VMEM_SHARED((N_PAD, HH), jnp.float32)]
        + [pltpu.SemaphoreType.DMA] * (2 * NBUF)
    ),
    compiler_params=pltpu.CompilerParams(use_tc_tiling_on_sc=False),
)

_sc_deg = pl.kernel(
    _sc_deg_body,
    out_type=jax.ShapeDtypeStruct((NC, N_PAD, 16), jnp.float32),
    mesh=_mesh,
    scratch_types=[
        pltpu.VMEM((CH, K), jnp.int32),
        pltpu.VMEM((K, 16), jnp.float32),
        pltpu.VMEM((WK, 16), jnp.float32),
        pltpu.VMEM_SHARED((N_PAD, 16), jnp.float32),
        pltpu.SemaphoreType.DMA,
    ],
    compiler_params=pltpu.CompilerParams(use_tc_tiling_on_sc=False),
)


def _dot_t(a, w):
    return lax.dot_general(a, w, (((1,), (1,)), ((), ())),
                           preferred_element_type=jnp.float32)


def _split_out(o0_ref, o1_ref, val):
    o0_ref[...] = val[:, :HH]
    o1_ref[...] = val[:, HH:]


def _proj_body(x_ref, w_ref, b_ref, o0_ref, o1_ref):
    t = jnp.maximum(_dot_t(x_ref[...], w_ref[...]) + b_ref[...], 0.0)
    _split_out(o0_ref, o1_ref, t)


def _proj(x, Win, b_in):
    full = lambda s: pl.BlockSpec(s, lambda i: (0, 0))
    half = pl.BlockSpec((_BLK, HH), lambda i: (i, 0))
    return pl.pallas_call(
        _proj_body,
        grid=(N // _BLK,),
        in_specs=[pl.BlockSpec((_BLK, D), lambda i: (i, 0)),
                  full((H, D)), full((1, H))],
        out_specs=(half, half),
        out_shape=(jax.ShapeDtypeStruct((N, HH), jnp.float32),
                   jax.ShapeDtypeStruct((N, HH), jnp.float32)),
    )(x, Win, b_in.reshape(1, -1))


def _layer_body(s0_ref, s1_ref, d_ref, h0_ref, h1_ref,
                wl_ref, bl_ref, wr_ref, ws_ref, bs_ref, g_ref, bt_ref,
                o0_ref, o1_ref):
    s = jnp.concatenate([s0_ref[0] + s0_ref[1], s1_ref[0] + s1_ref[1]],
                        axis=1)
    degc = d_ref[0] + d_ref[1]
    deg = degc[:, 0:1]
    mean = s * (1.0 / jnp.maximum(deg, 1.0))
    h = jnp.concatenate([h0_ref[...], h1_ref[...]], axis=1)
    t = _dot_t(mean, wl_ref[...]) + bl_ref[...] + _dot_t(h, wr_ref[...])
    t = t * g_ref[...] + bt_ref[...]
    res = jnp.maximum(t, 0.0) + _dot_t(h, ws_ref[...]) + bs_ref[...]
    _split_out(o0_ref, o1_ref, res)


def _layer(s0, s1, degs, h0, h1, Wl, bl, Wr, Ws, bs, gscale, beta):
    full = lambda s: pl.BlockSpec(s, lambda i: tuple(0 for _ in s))
    half = pl.BlockSpec((_BLK, HH), lambda i: (i, 0))
    shalf = pl.BlockSpec((NC, _BLK, HH), lambda i: (0, i, 0))
    return pl.pallas_call(
        _layer_body,
        grid=(N // _BLK,),
        in_specs=[
            shalf, shalf,
            pl.BlockSpec((NC, _BLK, 16), lambda i: (0, i, 0)),
            half, half,
            full((H, H)), full((1, H)), full((H, H)),
            full((H, H)), full((1, H)),
            full((1, H)), full((1, H)),
        ],
        out_specs=(half, half),
        out_shape=(jax.ShapeDtypeStruct((N, HH), jnp.float32),
                   jax.ShapeDtypeStruct((N, HH), jnp.float32)),
    )(s0, s1, degs, h0, h1, Wl, bl.reshape(1, -1), Wr, Ws,
      bs.reshape(1, -1), gscale.reshape(1, -1), beta.reshape(1, -1))


def _final_body(h0_ref, h1_ref, w1_ref, b1_ref, w2_ref, b2_ref, wo_ref,
                bo_ref, out_ref):
    h = jnp.concatenate([h0_ref[...], h1_ref[...]], axis=1)
    t = jnp.maximum(_dot_t(h, w1_ref[...]) + b1_ref[...], 0.0)
    t = _dot_t(t, w2_ref[...]) + b2_ref[...]
    o = _dot_t(t, wo_ref[...]) + bo_ref[...]
    m = jnp.max(o, axis=1, keepdims=True)
    lse = m + jnp.log(jnp.sum(jnp.exp(o - m), axis=1, keepdims=True))
    out_ref[...] = o - lse


def _final_stage(h0, h1, W1, b1, W2, b2, Wout, bout):
    full = lambda s: pl.BlockSpec(s, lambda i: (0, 0))
    half = pl.BlockSpec((_BLK, HH), lambda i: (i, 0))
    return pl.pallas_call(
        _final_body,
        grid=(N // _BLK,),
        in_specs=[
            half, half,
            full((2 * H, H)), full((1, 2 * H)),
            full((H, 2 * H)), full((1, H)),
            full((C, H)), full((1, C)),
        ],
        out_specs=pl.BlockSpec((_BLK, C), lambda i: (i, 0)),
        out_shape=jax.ShapeDtypeStruct((N, C), jnp.float32),
    )(h0, h1, W1, b1.reshape(1, -1), W2, b2.reshape(1, -1),
      Wout, bout.reshape(1, -1))


def kernel(x, edge_index, Win, b_in, convWl, convbl, convWr, skipW, skipb,
           gamma, beta, W1, b1, W2, b2, Wout, bout):
    srcp = edge_index[0].reshape(NW, CH, K)
    dstp = edge_index[1].reshape(NW, CH, K)
    z64 = jnp.zeros((WK, HH), jnp.float32)
    o16 = jnp.ones((K, 16), jnp.float32)
    z16 = jnp.zeros((WK, 16), jnp.float32)
    bn = 1.0 / jnp.sqrt(1.0 + 1e-5)

    h0, h1 = _proj(x, Win, b_in)
    degs = _sc_deg(dstp, o16, z16)
    for i in range(L):
        s0, s1 = _sc_agg(h0, h1, srcp, dstp, z64)
        h0, h1 = _layer(s0, s1, degs, h0, h1, convWl[i], convbl[i],
                        convWr[i], skipW[i], skipb[i], gamma[i] * bn, beta[i])
    return _final_stage(h0, h1, W1, b1, W2, b2, Wout, bout)


# speedup vs baseline: 1.9031x; 1.0158x over previous
import jax
import jax.numpy as jnp
from jax import lax
from jax.experimental import pallas as pl
from jax.experimental.pallas import tpu as pltpu
from jax.experimental.pallas import tpu_sc as plsc

N = 10000
E = 320000
D = 128
H = 128
HH = H // 2
C = 40
L = 4

NC = 2
NS = 16
NW = NC * NS

K = 80
CH = E // (K * NW)
NBUF = 12
LEAD = 6
CHE = CH - (CH % NBUF)
N_PAD = 10240
RPT = N_PAD // NS
WK = 128
WCH = RPT // WK

_BLK = 1000

_mesh = plsc.VectorSubcoreMesh(
    core_axis_name="c", subcore_axis_name="s", num_cores=NC, num_subcores=NS)


def _sc_agg_body(h0_hbm, h1_hbm, src_hbm, dst_hbm, z64_hbm,
                 s0_hbm, s1_hbm,
                 src_v, dst_v, zbuf, *bufs_and_sems):
    bufs = bufs_and_sems[:NBUF]
    acc_shr = bufs_and_sems[NBUF]
    gsems = bufs_and_sems[NBUF + 1:2 * NBUF + 1]
    ssems = bufs_and_sems[2 * NBUF + 1:]
    cid = lax.axis_index("c")
    sid = lax.axis_index("s")
    wid = cid * NS + sid
    base = sid * RPT
    pltpu.sync_copy(src_hbm.at[wid], src_v)
    pltpu.sync_copy(dst_hbm.at[wid], dst_v)
    pltpu.sync_copy(z64_hbm, zbuf)

    for cp, (h_hbm, out_hbm) in enumerate(((h0_hbm, s0_hbm),
                                           (h1_hbm, s1_hbm))):
        for w in range(WCH):
            pltpu.sync_copy(zbuf, acc_shr.at[pl.ds(base + w * WK, WK)])
        plsc.subcore_barrier()

        for m in range(LEAD):
            pltpu.async_copy(h_hbm.at[src_v.at[m]], bufs[m], gsems[m])

        @pl.loop(0, CHE, step=NBUF)
        def _(j):
            for b in range(NBUF):
                jj = j + b
                pltpu.make_async_copy(h_hbm.at[src_v.at[jj]], bufs[b],
                                      gsems[b]).wait()
                pltpu.async_copy(bufs[b], acc_shr.at[dst_v.at[jj]],
                                 ssems[b], add=True)
                nb = (b + LEAD) % NBUF

                @pl.when(jj >= LEAD)
                def _():
                    pltpu.make_async_copy(
                        bufs[nb], acc_shr.at[dst_v.at[jj - LEAD]],
                        ssems[nb]).wait()

                @pl.when(jj + LEAD < CH)
                def _():
                    pltpu.async_copy(h_hbm.at[src_v.at[jj + LEAD]],
                                     bufs[nb], gsems[nb])

        for jj in range(CHE - LEAD, CHE):
            b = jj % NBUF
            pltpu.make_async_copy(bufs[b], acc_shr.at[dst_v.at[jj]],
                                  ssems[b]).wait()
        for jj in range(CHE, CH):
            b = jj % NBUF
            if jj >= CHE + LEAD:
                pltpu.async_copy(h_hbm.at[src_v.at[jj]], bufs[b],
                                 gsems[b])
            pltpu.make_async_copy(h_hbm.at[src_v.at[jj]], bufs[b],
                                  gsems[b]).wait()
            pltpu.sync_copy(bufs[b], acc_shr.at[dst_v.at[jj]], add=True)

        plsc.subcore_barrier()
        for w in range(WCH):
            r = base + w * WK
            pltpu.sync_copy(acc_shr.at[pl.ds(r, WK)], zbuf)
            pltpu.sync_copy(zbuf, out_hbm.at[cid, pl.ds(r, WK)])
        if cp == 0:
            pltpu.sync_copy(z64_hbm, zbuf)
            plsc.subcore_barrier()


def _sc_deg_body(dst_hbm, o16_hbm, z16_hbm,
                 deg_hbm,
                 dst_v, ones_v, d16_v,
                 dacc_shr, sem0):
    cid = lax.axis_index("c")
    sid = lax.axis_index("s")
    wid = cid * NS + sid
    base = sid * RPT
    pltpu.sync_copy(dst_hbm.at[wid], dst_v)
    pltpu.sync_copy(z16_hbm, d16_v)
    for w in range(WCH):
        pltpu.sync_copy(d16_v, dacc_shr.at[pl.ds(base + w * WK, WK)])
    pltpu.sync_copy(o16_hbm, ones_v)
    plsc.subcore_barrier()

    @pl.loop(0, CH)
    def _(j):
        pltpu.sync_copy(ones_v, dacc_shr.at[dst_v.at[j]], add=True)

    plsc.subcore_barrier()
    for w in range(WCH):
        r = base + w * WK
        pltpu.sync_copy(dacc_shr.at[pl.ds(r, WK)], d16_v)
        pltpu.sync_copy(d16_v, deg_hbm.at[cid, pl.ds(r, WK)])


_sc_agg = pl.kernel(
    _sc_agg_body,
    out_type=(jax.ShapeDtypeStruct((NC, N_PAD, HH), jnp.float32),
              jax.ShapeDtypeStruct((NC, N_PAD, HH), jnp.float32)),
    mesh=_mesh,
    scratch_types=(
        [pltpu.VMEM((CH, K), jnp.int32),
         pltpu.VMEM((CH, K), jnp.int32),
         pltpu.VMEM((WK, HH), jnp.float32)]
        + [pltpu.VMEM((K, HH), jnp.float32)] * NBUF
        + [pltpu.VMEM_SHARED((N_PAD, HH), jnp.float32)]
        + [pltpu.SemaphoreType.DMA] * (2 * NBUF)
    ),
    compiler_params=pltpu.CompilerParams(use_tc_tiling_on_sc=False),
)

_sc_deg = pl.kernel(
    _sc_deg_body,
    out_type=jax.ShapeDtypeStruct((NC, N_PAD, 16), jnp.float32),
    mesh=_mesh,
    scratch_types=[
        pltpu.VMEM((CH, K), jnp.int32),
        pltpu.VMEM((K, 16), jnp.float32),
        pltpu.VMEM((WK, 16), jnp.float32),
        pltpu.VMEM_SHARED((N_PAD, 16), jnp.float32),
        pltpu.SemaphoreType.DMA,
    ],
    compiler_params=pltpu.CompilerParams(use_tc_tiling_on_sc=False),
)


def _dot_t(a, w):
    return lax.dot_general(a, w, (((1,), (1,)), ((), ())),
                           preferred_element_type=jnp.float32)


def _split_out(o0_ref, o1_ref, val):
    o0_ref[...] = val[:, :HH]
    o1_ref[...] = val[:, HH:]


def _proj_body(x_ref, w_ref, b_ref, o0_ref, o1_ref):
    t = jnp.maximum(_dot_t(x_ref[...], w_ref[...]) + b_ref[...], 0.0)
    _split_out(o0_ref, o1_ref, t)


def _proj(x, Win, b_in):
    full = lambda s: pl.BlockSpec(s, lambda i: (0, 0))
    half = pl.BlockSpec((_BLK, HH), lambda i: (i, 0))
    return pl.pallas_call(
        _proj_body,
        grid=(N // _BLK,),
        in_specs=[pl.BlockSpec((_BLK, D), lambda i: (i, 0)),
                  full((H, D)), full((1, H))],
        out_specs=(half, half),
        out_shape=(jax.ShapeDtypeStruct((N, HH), jnp.float32),
                   jax.ShapeDtypeStruct((N, HH), jnp.float32)),
    )(x, Win, b_in.reshape(1, -1))


def _layer_body(s0_ref, s1_ref, d_ref, h0_ref, h1_ref,
                wl_ref, bl_ref, wr_ref, ws_ref, bs_ref, g_ref, bt_ref,
                o0_ref, o1_ref):
    s = jnp.concatenate([s0_ref[0] + s0_ref[1], s1_ref[0] + s1_ref[1]],
                        axis=1)
    degc = d_ref[0] + d_ref[1]
    deg = degc[:, 0:1]
    mean = s * (1.0 / jnp.maximum(deg, 1.0))
    h = jnp.concatenate([h0_ref[...], h1_ref[...]], axis=1)
    t = _dot_t(mean, wl_ref[...]) + bl_ref[...] + _dot_t(h, wr_ref[...])
    t = t * g_ref[...] + bt_ref[...]
    res = jnp.maximum(t, 0.0) + _dot_t(h, ws_ref[...]) + bs_ref[...]
    _split_out(o0_ref, o1_ref, res)


def _layer(s0, s1, degs, h0, h1, Wl, bl, Wr, Ws, bs, gscale, beta):
    full = lambda s: pl.BlockSpec(s, lambda i: tuple(0 for _ in s))
    half = pl.BlockSpec((_BLK, HH), lambda i: (i, 0))
    shalf = pl.BlockSpec((NC, _BLK, HH), lambda i: (0, i, 0))
    return pl.pallas_call(
        _layer_body,
        grid=(N // _BLK,),
        in_specs=[
            shalf, shalf,
            pl.BlockSpec((NC, _BLK, 16), lambda i: (0, i, 0)),
            half, half,
            full((H, H)), full((1, H)), full((H, H)),
            full((H, H)), full((1, H)),
            full((1, H)), full((1, H)),
        ],
        out_specs=(half, half),
        out_shape=(jax.ShapeDtypeStruct((N, HH), jnp.float32),
                   jax.ShapeDtypeStruct((N, HH), jnp.float32)),
    )(s0, s1, degs, h0, h1, Wl, bl.reshape(1, -1), Wr, Ws,
      bs.reshape(1, -1), gscale.reshape(1, -1), beta.reshape(1, -1))


def _final_body(h0_ref, h1_ref, w1_ref, b1_ref, w2_ref, b2_ref, wo_ref,
                bo_ref, out_ref):
    h = jnp.concatenate([h0_ref[...], h1_ref[...]], axis=1)
    t = jnp.maximum(_dot_t(h, w1_ref[...]) + b1_ref[...], 0.0)
    t = _dot_t(t, w2_ref[...]) + b2_ref[...]
    o = _dot_t(t, wo_ref[...]) + bo_ref[...]
    m = jnp.max(o, axis=1, keepdims=True)
    lse = m + jnp.log(jnp.sum(jnp.exp(o - m), axis=1, keepdims=True))
    out_ref[...] = o - lse


def _final_stage(h0, h1, W1, b1, W2, b2, Wout, bout):
    full = lambda s: pl.BlockSpec(s, lambda i: (0, 0))
    half = pl.BlockSpec((_BLK, HH), lambda i: (i, 0))
    return pl.pallas_call(
        _final_body,
        grid=(N // _BLK,),
        in_specs=[
            half, half,
            full((2 * H, H)), full((1, 2 * H)),
            full((H, 2 * H)), full((1, H)),
            full((C, H)), full((1, C)),
        ],
        out_specs=pl.BlockSpec((_BLK, C), lambda i: (i, 0)),
        out_shape=jax.ShapeDtypeStruct((N, C), jnp.float32),
    )(h0, h1, W1, b1.reshape(1, -1), W2, b2.reshape(1, -1),
      Wout, bout.reshape(1, -1))


def kernel(x, edge_index, Win, b_in, convWl, convbl, convWr, skipW, skipb,
           gamma, beta, W1, b1, W2, b2, Wout, bout):
    srcp = edge_index[0].reshape(NW, CH, K)
    dstp = edge_index[1].reshape(NW, CH, K)
    z64 = jnp.zeros((WK, HH), jnp.float32)
    o16 = jnp.ones((K, 16), jnp.float32)
    z16 = jnp.zeros((WK, 16), jnp.float32)
    bn = 1.0 / jnp.sqrt(1.0 + 1e-5)

    h0, h1 = _proj(x, Win, b_in)
    degs = _sc_deg(dstp, o16, z16)
    for i in range(L):
        s0, s1 = _sc_agg(h0, h1, srcp, dstp, z64)
        h0, h1 = _layer(s0, s1, degs, h0, h1, convWl[i], convbl[i],
                        convWr[i], skipW[i], skipb[i], gamma[i] * bn, beta[i])
    return _final_stage(h0, h1, W1, b1, W2, b2, Wout, bout)
